# Initial kernel scaffold; baseline (speedup 1.0000x reference)
import jax
import jax.numpy as jnp
from jax.experimental import pallas as pl


def kernel(f_atoms, f_bonds, a2b, b2a, b2revb, W_in_w, W_in_b, W_msg_w, W_msg_b, W_atom_w, W_atom_b, node_w, node_b, edge_w, edge_b):
    raise NotImplementedError("write your pallas kernel here")


# speedup vs baseline: 1.6913x; 1.6913x over previous
import functools

import jax
import jax.numpy as jnp
from jax import lax
from jax.experimental import pallas as pl
from jax.experimental.pallas import tpu as pltpu
from jax.experimental.pallas import tpu_sc as plsc

NE = 320000
NA = 10000
H = 128
DIN = 144
NC, NS = 2, 16
NW = NC * NS
C = 128
NCHUNK = NE // C
CPW = (NCHUNK + NW - 1) // NW
RPT = 624
TAIL = NA - RPT * NS

_MESH = plsc.VectorSubcoreMesh(core_axis_name="c", subcore_axis_name="s")


def _sc_first_body(b2a_h, b2revb_h, h_h, z_h, dest_h, spart_h,
                   table, hbuf, ir, idst, sem):
    cid = lax.axis_index("c")
    sid = lax.axis_index("s")
    wid = sid * NC + cid
    pltpu.sync_copy(z_h, table.at[pl.ds(sid * RPT, RPT)])

    @pl.when(sid == 0)
    def _():
        pltpu.sync_copy(z_h.at[pl.ds(0, TAIL)], table.at[pl.ds(NS * RPT, TAIL)])

    plsc.subcore_barrier()

    def body(i, carry):
        ch = wid + i * NW

        @pl.when(ch < NCHUNK)
        def _():
            base = ch * C
            pltpu.sync_copy(b2revb_h.at[pl.ds(base, C)], ir)
            pltpu.async_copy(b2a_h.at[ir], idst, sem).wait()
            pltpu.sync_copy(idst, dest_h.at[pl.ds(base, C)])
            pltpu.sync_copy(h_h.at[pl.ds(base, C)], hbuf)
            pltpu.sync_copy(hbuf, table.at[idst], add=True)

        return carry

    lax.fori_loop(0, CPW, body, 0)
    plsc.subcore_barrier()
    rows = pl.ds(sid * RPT, RPT)
    pltpu.sync_copy(table.at[rows], spart_h.at[cid].at[rows])

    @pl.when(sid == 0)
    def _():
        tail = pl.ds(NS * RPT, TAIL)
        pltpu.sync_copy(table.at[tail], spart_h.at[cid].at[tail])


_sc_first = functools.partial(
    pl.kernel,
    out_type=(jax.ShapeDtypeStruct((NE,), jnp.int32),
              jax.ShapeDtypeStruct((NC, NA, H), jnp.float32)),
    mesh=_MESH,
    scratch_types=[
        pltpu.VMEM_SHARED((NA, H), jnp.float32),
        pltpu.VMEM((C, H), jnp.float32),
        pltpu.VMEM((C,), jnp.int32),
        pltpu.VMEM((C,), jnp.int32),
        pltpu.SemaphoreType.DMA,
    ],
)(_sc_first_body)


def _sc_step_body(h_h, sw_h, hw_h, b2a_h, b2revb_h, dest_h, z_h,
                  hnew_h, spart_h,
                  table, hbuf, swbuf, hwbuf, ia, ir, idst, sem1, sem2):
    cid = lax.axis_index("c")
    sid = lax.axis_index("s")
    wid = sid * NC + cid
    pltpu.sync_copy(z_h, table.at[pl.ds(sid * RPT, RPT)])

    @pl.when(sid == 0)
    def _():
        pltpu.sync_copy(z_h.at[pl.ds(0, TAIL)], table.at[pl.ds(NS * RPT, TAIL)])

    plsc.subcore_barrier()

    def body(i, carry):
        ch = wid + i * NW

        @pl.when(ch < NCHUNK)
        def _():
            base = ch * C
            pltpu.sync_copy(b2a_h.at[pl.ds(base, C)], ia)
            pltpu.sync_copy(b2revb_h.at[pl.ds(base, C)], ir)
            pltpu.sync_copy(dest_h.at[pl.ds(base, C)], idst)
            cp1 = pltpu.async_copy(sw_h.at[ia], swbuf, sem1)
            cp2 = pltpu.async_copy(hw_h.at[ir], hwbuf, sem2)
            pltpu.sync_copy(h_h.at[pl.ds(base, C)], hbuf)
            cp1.wait()
            cp2.wait()

            def comb(r, cc):
                for k in range(H // 16):
                    sl = pl.ds(k * 16, 16)
                    v = hbuf[r, sl] + swbuf[r, sl] - hwbuf[r, sl]
                    hbuf[r, sl] = jnp.maximum(v, 0.0)
                return cc

            lax.fori_loop(0, C, comb, 0)
            pltpu.sync_copy(hbuf, hnew_h.at[pl.ds(base, C)])
            pltpu.sync_copy(hbuf, table.at[idst], add=True)

        return carry

    lax.fori_loop(0, CPW, body, 0)
    plsc.subcore_barrier()
    rows = pl.ds(sid * RPT, RPT)
    pltpu.sync_copy(table.at[rows], spart_h.at[cid].at[rows])

    @pl.when(sid == 0)
    def _():
        tail = pl.ds(NS * RPT, TAIL)
        pltpu.sync_copy(table.at[tail], spart_h.at[cid].at[tail])


_sc_step = functools.partial(
    pl.kernel,
    out_type=(jax.ShapeDtypeStruct((NE, H), jnp.float32),
              jax.ShapeDtypeStruct((NC, NA, H), jnp.float32)),
    mesh=_MESH,
    scratch_types=[
        pltpu.VMEM_SHARED((NA, H), jnp.float32),
        pltpu.VMEM((C, H), jnp.float32),
        pltpu.VMEM((C, H), jnp.float32),
        pltpu.VMEM((C, H), jnp.float32),
        pltpu.VMEM((C,), jnp.int32),
        pltpu.VMEM((C,), jnp.int32),
        pltpu.VMEM((C,), jnp.int32),
        pltpu.SemaphoreType.DMA,
        pltpu.SemaphoreType.DMA,
    ],
)(_sc_step_body)


def _tc_in_body(fb, win, bin_, wmsg, h_o, hw_o):
    h = jnp.maximum(
        jnp.dot(fb[...], win[...], preferred_element_type=jnp.float32)
        + bin_[...], 0.0)
    h_o[...] = h
    hw_o[...] = jnp.dot(h, wmsg[...], preferred_element_type=jnp.float32)


def _tc_in(f_bonds, W_in_w, bin2, W_msg_w):
    R = 1280
    return pl.pallas_call(
        _tc_in_body,
        grid=(NE // R,),
        in_specs=[pl.BlockSpec((R, DIN), lambda i: (i, 0)),
                  pl.BlockSpec((DIN, H), lambda i: (0, 0)),
                  pl.BlockSpec((1, H), lambda i: (0, 0)),
                  pl.BlockSpec((H, H), lambda i: (0, 0))],
        out_specs=[pl.BlockSpec((R, H), lambda i: (i, 0)),
                   pl.BlockSpec((R, H), lambda i: (i, 0))],
        out_shape=[jax.ShapeDtypeStruct((NE, H), jnp.float32),
                   jax.ShapeDtypeStruct((NE, H), jnp.float32)],
    )(f_bonds, W_in_w, bin2, W_msg_w)


def _tc_hw_body(h, wmsg, hw_o):
    hw_o[...] = jnp.dot(h[...], wmsg[...], preferred_element_type=jnp.float32)


def _tc_hw(h, W_msg_w):
    R = 2000
    return pl.pallas_call(
        _tc_hw_body,
        grid=(NE // R,),
        in_specs=[pl.BlockSpec((R, H), lambda i: (i, 0)),
                  pl.BlockSpec((H, H), lambda i: (0, 0))],
        out_specs=pl.BlockSpec((R, H), lambda i: (i, 0)),
        out_shape=jax.ShapeDtypeStruct((NE, H), jnp.float32),
    )(h, W_msg_w)


def _tc_sw_body(s0, s1, wmsg, bmsg, sw_o):
    s = s0[...] + s1[...]
    sw_o[...] = (jnp.dot(s, wmsg[...], preferred_element_type=jnp.float32)
                 + bmsg[...])


def _tc_sw(s0, s1, W_msg_w, bmsg2):
    return pl.pallas_call(
        _tc_sw_body,
        out_shape=jax.ShapeDtypeStruct((NA, H), jnp.float32),
    )(s0, s1, W_msg_w, bmsg2)


def _tc_atom_body(s0, s1, fa, wat, bat, nw, nb, hatom_o, node_o):
    s = s0[...] + s1[...]
    wa = wat[...]
    h_atom = jnp.maximum(
        jnp.dot(s, wa[0:H, :], preferred_element_type=jnp.float32)
        + jnp.dot(fa[...], wa[H:2 * H, :], preferred_element_type=jnp.float32)
        + bat[...], 0.0)
    hatom_o[...] = h_atom
    node_o[...] = (jnp.dot(h_atom, nw[...], preferred_element_type=jnp.float32)
                   + nb[...])


def _tc_atom(s0, s1, f_atoms, W_atom_w, bat2, node_w, nb2):
    return pl.pallas_call(
        _tc_atom_body,
        out_shape=[jax.ShapeDtypeStruct((NA, H), jnp.float32),
                   jax.ShapeDtypeStruct((NA, H), jnp.float32)],
    )(s0, s1, f_atoms, W_atom_w, bat2, node_w, nb2)


def _tc_edge_body(h, ew, eb, out):
    out[...] = (jnp.dot(h[...], ew[...], preferred_element_type=jnp.float32)
                + eb[...])


def _tc_edge(h, edge_w, eb2):
    R = 2000
    return pl.pallas_call(
        _tc_edge_body,
        grid=(NE // R,),
        in_specs=[pl.BlockSpec((R, H), lambda i: (i, 0)),
                  pl.BlockSpec((H, DIN), lambda i: (0, 0)),
                  pl.BlockSpec((1, DIN), lambda i: (0, 0))],
        out_specs=pl.BlockSpec((R, DIN), lambda i: (i, 0)),
        out_shape=jax.ShapeDtypeStruct((NE, DIN), jnp.float32),
    )(h, edge_w, eb2)


def kernel(f_atoms, f_bonds, a2b, b2a, b2revb,
           W_in_w, W_in_b, W_msg_w, W_msg_b,
           W_atom_w, W_atom_b, node_w, node_b, edge_w, edge_b):
    del a2b
    b2a = b2a.astype(jnp.int32)
    b2revb = b2revb.astype(jnp.int32)
    bin2 = W_in_b.reshape(1, H)
    bmsg2 = W_msg_b.reshape(1, H)
    bat2 = W_atom_b.reshape(1, H)
    nb2 = node_b.reshape(1, H)
    eb2 = edge_b.reshape(1, DIN)
    zrows = jnp.zeros((RPT, H), jnp.float32)

    h, hw = _tc_in(f_bonds, W_in_w, bin2, W_msg_w)
    dest, spart = _sc_first(b2a, b2revb, h, zrows)
    for step in range(3):
        sw = _tc_sw(spart[0], spart[1], W_msg_w, bmsg2)
        h, spart = _sc_step(h, sw, hw, b2a, b2revb, dest, zrows)
        if step < 2:
            hw = _tc_hw(h, W_msg_w)

    h_atom, node_pred = _tc_atom(spart[0], spart[1], f_atoms,
                                 W_atom_w, bat2, node_w, nb2)
    edge_pred = _tc_edge(h, edge_w, eb2)
    return (node_pred, edge_pred, h_atom)


# speedup vs baseline: 1.8364x; 1.0858x over previous
import functools

import jax
import jax.numpy as jnp
from jax import lax
from jax.experimental import pallas as pl
from jax.experimental.pallas import tpu as pltpu
from jax.experimental.pallas import tpu_sc as plsc

NE = 320000
NA = 10000
H = 128
DIN = 144
NC, NS = 2, 16
NW = NC * NS
C = 80
NCHUNK = NE // C
CPW = NCHUNK // NW
CS = 40
NCHUNKS = NE // CS
CPWS = NCHUNKS // NW
RPT = 624
TAIL = NA - RPT * NS

_MESH = plsc.VectorSubcoreMesh(core_axis_name="c", subcore_axis_name="s")


def _sc_first_body(b2a_h, b2revb_h, h_h, z_h, dest_h, spart_h,
                   table, hbuf, ir, idst, sem):
    cid = lax.axis_index("c")
    sid = lax.axis_index("s")
    wid = sid * NC + cid
    pltpu.sync_copy(z_h, table.at[pl.ds(sid * RPT, RPT)])

    @pl.when(sid == 0)
    def _():
        pltpu.sync_copy(z_h.at[pl.ds(0, TAIL)], table.at[pl.ds(NS * RPT, TAIL)])

    plsc.subcore_barrier()

    def body(i, carry):
        base = (wid * CPW + i) * C
        pltpu.sync_copy(b2revb_h.at[pl.ds(base, C)], ir)
        pltpu.async_copy(b2a_h.at[ir], idst, sem).wait()
        pltpu.sync_copy(idst, dest_h.at[pl.ds(base, C)])
        pltpu.sync_copy(h_h.at[pl.ds(base, C)], hbuf)
        pltpu.sync_copy(hbuf, table.at[idst], add=True)
        return carry

    lax.fori_loop(0, CPW, body, 0)
    plsc.subcore_barrier()
    rows = pl.ds(sid * RPT, RPT)
    pltpu.sync_copy(table.at[rows], spart_h.at[cid].at[rows])

    @pl.when(sid == 0)
    def _():
        tail = pl.ds(NS * RPT, TAIL)
        pltpu.sync_copy(table.at[tail], spart_h.at[cid].at[tail])


_sc_first = functools.partial(
    pl.kernel,
    out_type=(jax.ShapeDtypeStruct((NE,), jnp.int32),
              jax.ShapeDtypeStruct((NC, NA, H), jnp.float32)),
    mesh=_MESH,
    scratch_types=[
        pltpu.VMEM_SHARED((NA, H), jnp.float32),
        pltpu.VMEM((C, H), jnp.float32),
        pltpu.VMEM((C,), jnp.int32),
        pltpu.VMEM((C,), jnp.int32),
        pltpu.SemaphoreType.DMA,
    ],
)(_sc_first_body)


def _sc_step_body(h_h, sw_h, hw_h, b2a_h, b2revb_h, dest_h, z_h,
                  hnew_h, spart_h,
                  table,
                  hbuf0, hbuf1, obuf0, obuf1, swbuf0, swbuf1,
                  hwbuf0, hwbuf1, ia0, ia1, ir0, ir1, id0, id1,
                  si0, si1, sg0, sg1, so0, so1, sd0, sd1):
    cid = lax.axis_index("c")
    sid = lax.axis_index("s")
    wid = sid * NC + cid
    base0 = wid * CPWS * CS
    pltpu.sync_copy(z_h, table.at[pl.ds(sid * RPT, RPT)])

    @pl.when(sid == 0)
    def _():
        pltpu.sync_copy(z_h.at[pl.ds(0, TAIL)], table.at[pl.ds(NS * RPT, TAIL)])

    plsc.subcore_barrier()

    slot = [
        dict(h=hbuf0, o=obuf0, sw=swbuf0, hw=hwbuf0, ia=ia0, ir=ir0,
             idst=id0, si=si0, sg=sg0, so=so0, sd=sd0),
        dict(h=hbuf1, o=obuf1, sw=swbuf1, hw=hwbuf1, ia=ia1, ir=ir1,
             idst=id1, si=si1, sg=sg1, so=so1, sd=sd1),
    ]

    def prefetch(i, s):
        b = base0 + i * CS
        pltpu.async_copy(b2a_h.at[pl.ds(b, CS)], s["ia"], s["si"])
        pltpu.async_copy(b2revb_h.at[pl.ds(b, CS)], s["ir"], s["si"])
        pltpu.async_copy(h_h.at[pl.ds(b, CS)], s["h"], s["si"])

    def wait_in(i, s):
        b = base0 + i * CS
        pltpu.make_async_copy(b2a_h.at[pl.ds(b, CS)], s["ia"], s["si"]).wait()
        pltpu.make_async_copy(b2revb_h.at[pl.ds(b, CS)], s["ir"], s["si"]).wait()
        pltpu.make_async_copy(h_h.at[pl.ds(b, CS)], s["h"], s["si"]).wait()

    def idst_load(i, s):
        pltpu.async_copy(dest_h.at[pl.ds(base0 + i * CS, CS)], s["idst"], s["sd"])

    def wait_idst(i, s):
        pltpu.make_async_copy(dest_h.at[pl.ds(base0 + i * CS, CS)],
                              s["idst"], s["sd"]).wait()

    def gathers(s):
        pltpu.async_copy(sw_h.at[s["ia"]], s["sw"], s["sg"])
        pltpu.async_copy(hw_h.at[s["ir"]], s["hw"], s["sg"])

    def wait_g(s):
        pltpu.make_async_copy(sw_h.at[s["ia"]], s["sw"], s["sg"]).wait()
        pltpu.make_async_copy(hw_h.at[s["ir"]], s["hw"], s["sg"]).wait()

    def writeback(i, s):
        b = base0 + i * CS
        pltpu.async_copy(s["o"], hnew_h.at[pl.ds(b, CS)], s["so"])
        pltpu.sync_copy(s["o"], table.at[s["idst"]], add=True)

    def wait_out(i, s):
        b = base0 + i * CS
        pltpu.make_async_copy(s["o"], hnew_h.at[pl.ds(b, CS)], s["so"]).wait()

    def compute(s):
        hb, ob, sb, wb = s["h"], s["o"], s["sw"], s["hw"]

        def comb(r, cc):
            for k in range(H // 16):
                sl = pl.ds(k * 16, 16)
                ob[r, sl] = jnp.maximum(hb[r, sl] + sb[r, sl] - wb[r, sl], 0.0)
            return cc

        lax.fori_loop(0, CS, comb, 0)

    def emit(i, sl, last):
        if not last:
            other = slot[1] if sl is slot[0] else slot[0]
            wait_in(i + 1, other)
            gathers(other)
        wait_g(sl)

        def drain():
            wait_out(i - 2, sl)
            idst_load(i, sl)

        if isinstance(i, int):
            if i >= 2:
                drain()
        else:
            pl.when(i >= 2)(drain)
        compute(sl)
        wait_idst(i, sl)
        writeback(i, sl)
        if not last:
            def pre():
                prefetch(i + 2, sl)

            if isinstance(i, int):
                if i + 2 < CPWS:
                    pre()
            else:
                pl.when(i + 2 < CPWS)(pre)

    prefetch(0, slot[0])
    idst_load(0, slot[0])
    prefetch(1, slot[1])
    idst_load(1, slot[1])
    wait_in(0, slot[0])
    gathers(slot[0])

    def pair(p, carry):
        emit(2 * p, slot[0], last=False)
        emit(2 * p + 1, slot[1], last=False)
        return carry

    lax.fori_loop(0, CPWS // 2 - 1, pair, 0)
    emit(CPWS - 2, slot[0], last=False)
    emit(CPWS - 1, slot[1], last=True)

    wait_out(CPWS - 2, slot[(CPWS - 2) % 2])
    wait_out(CPWS - 1, slot[(CPWS - 1) % 2])

    plsc.subcore_barrier()
    rows = pl.ds(sid * RPT, RPT)
    pltpu.sync_copy(table.at[rows], spart_h.at[cid].at[rows])

    @pl.when(sid == 0)
    def _():
        tail = pl.ds(NS * RPT, TAIL)
        pltpu.sync_copy(table.at[tail], spart_h.at[cid].at[tail])


_sc_step = functools.partial(
    pl.kernel,
    out_type=(jax.ShapeDtypeStruct((NE, H), jnp.float32),
              jax.ShapeDtypeStruct((NC, NA, H), jnp.float32)),
    mesh=_MESH,
    scratch_types=(
        [pltpu.VMEM_SHARED((NA, H), jnp.float32)]
        + [pltpu.VMEM((CS, H), jnp.float32)] * 8
        + [pltpu.VMEM((CS,), jnp.int32)] * 6
        + [pltpu.SemaphoreType.DMA] * 8
    ),
)(_sc_step_body)


def _tc_in_body(fb, win, bin_, wmsg, h_o, hw_o):
    h = jnp.maximum(
        jnp.dot(fb[...], win[...], preferred_element_type=jnp.float32)
        + bin_[...], 0.0)
    h_o[...] = h
    hw_o[...] = jnp.dot(h, wmsg[...], preferred_element_type=jnp.float32)


def _tc_in(f_bonds, W_in_w, bin2, W_msg_w):
    R = 1280
    return pl.pallas_call(
        _tc_in_body,
        grid=(NE // R,),
        in_specs=[pl.BlockSpec((R, DIN), lambda i: (i, 0)),
                  pl.BlockSpec((DIN, H), lambda i: (0, 0)),
                  pl.BlockSpec((1, H), lambda i: (0, 0)),
                  pl.BlockSpec((H, H), lambda i: (0, 0))],
        out_specs=[pl.BlockSpec((R, H), lambda i: (i, 0)),
                   pl.BlockSpec((R, H), lambda i: (i, 0))],
        out_shape=[jax.ShapeDtypeStruct((NE, H), jnp.float32),
                   jax.ShapeDtypeStruct((NE, H), jnp.float32)],
    )(f_bonds, W_in_w, bin2, W_msg_w)


def _tc_hw_body(h, wmsg, hw_o):
    hw_o[...] = jnp.dot(h[...], wmsg[...], preferred_element_type=jnp.float32)


def _tc_hw(h, W_msg_w):
    R = 2000
    return pl.pallas_call(
        _tc_hw_body,
        grid=(NE // R,),
        in_specs=[pl.BlockSpec((R, H), lambda i: (i, 0)),
                  pl.BlockSpec((H, H), lambda i: (0, 0))],
        out_specs=pl.BlockSpec((R, H), lambda i: (i, 0)),
        out_shape=jax.ShapeDtypeStruct((NE, H), jnp.float32),
    )(h, W_msg_w)


def _tc_sw_body(s0, s1, wmsg, bmsg, sw_o):
    s = s0[...] + s1[...]
    sw_o[...] = (jnp.dot(s, wmsg[...], preferred_element_type=jnp.float32)
                 + bmsg[...])


def _tc_sw(s0, s1, W_msg_w, bmsg2):
    return pl.pallas_call(
        _tc_sw_body,
        out_shape=jax.ShapeDtypeStruct((NA, H), jnp.float32),
    )(s0, s1, W_msg_w, bmsg2)


def _tc_atom_body(s0, s1, fa, was, waf, bat, nw, nb, hatom_o, node_o):
    s = s0[...] + s1[...]
    h_atom = jnp.maximum(
        jnp.dot(s, was[...], preferred_element_type=jnp.float32)
        + jnp.dot(fa[...], waf[...], preferred_element_type=jnp.float32)
        + bat[...], 0.0)
    hatom_o[...] = h_atom
    node_o[...] = (jnp.dot(h_atom, nw[...], preferred_element_type=jnp.float32)
                   + nb[...])


def _tc_atom(s0, s1, f_atoms, wa_s, wa_f, bat2, node_w, nb2):
    return pl.pallas_call(
        _tc_atom_body,
        out_shape=[jax.ShapeDtypeStruct((NA, H), jnp.float32),
                   jax.ShapeDtypeStruct((NA, H), jnp.float32)],
    )(s0, s1, f_atoms, wa_s, wa_f, bat2, node_w, nb2)


def _tc_edge_body(h, ew, eb, out):
    out[...] = (jnp.dot(h[...], ew[...], preferred_element_type=jnp.float32)
                + eb[...])


def _tc_edge(h, edge_w, eb2):
    R = 2000
    return pl.pallas_call(
        _tc_edge_body,
        grid=(NE // R,),
        in_specs=[pl.BlockSpec((R, H), lambda i: (i, 0)),
                  pl.BlockSpec((H, DIN), lambda i: (0, 0)),
                  pl.BlockSpec((1, DIN), lambda i: (0, 0))],
        out_specs=pl.BlockSpec((R, DIN), lambda i: (i, 0)),
        out_shape=jax.ShapeDtypeStruct((NE, DIN), jnp.float32),
    )(h, edge_w, eb2)


def kernel(f_atoms, f_bonds, a2b, b2a, b2revb,
           W_in_w, W_in_b, W_msg_w, W_msg_b,
           W_atom_w, W_atom_b, node_w, node_b, edge_w, edge_b):
    del a2b
    b2a = b2a.astype(jnp.int32)
    b2revb = b2revb.astype(jnp.int32)
    bin2 = W_in_b.reshape(1, H)
    bmsg2 = W_msg_b.reshape(1, H)
    bat2 = W_atom_b.reshape(1, H)
    nb2 = node_b.reshape(1, H)
    eb2 = edge_b.reshape(1, DIN)
    wa_s = W_atom_w[0:H, :]
    wa_f = W_atom_w[H:2 * H, :]
    zrows = jnp.zeros((RPT, H), jnp.float32)

    h, hw = _tc_in(f_bonds, W_in_w, bin2, W_msg_w)
    dest, spart = _sc_first(b2a, b2revb, h, zrows)
    for step in range(3):
        sw = _tc_sw(spart[0], spart[1], W_msg_w, bmsg2)
        h, spart = _sc_step(h, sw, hw, b2a, b2revb, dest, zrows)
        if step < 2:
            hw = _tc_hw(h, W_msg_w)

    h_atom, node_pred = _tc_atom(spart[0], spart[1], f_atoms,
                                 wa_s, wa_f, bat2, node_w, nb2)
    edge_pred = _tc_edge(h, edge_w, eb2)
    return (node_pred, edge_pred, h_atom)


# speedup vs baseline: 1.8804x; 1.0239x over previous
import functools

import jax
import jax.numpy as jnp
from jax import lax
from jax.experimental import pallas as pl
from jax.experimental.pallas import tpu as pltpu
from jax.experimental.pallas import tpu_sc as plsc

NE = 320000
NA = 10000
H = 128
DIN = 144
NC, NS = 2, 16
NW = NC * NS
C = 128
NCHUNK = NE // C
CPW = (NCHUNK + NW - 1) // NW
CS = 40
NCHUNKS = NE // CS
CPWS = NCHUNKS // NW
RPT = 624
TAIL = NA - RPT * NS

_MESH = plsc.VectorSubcoreMesh(core_axis_name="c", subcore_axis_name="s")


def _sc_first_body(b2a_h, b2revb_h, h_h, z_h, dest_h, spart_h,
                   table, hbuf, ir, idst, sem):
    cid = lax.axis_index("c")
    sid = lax.axis_index("s")
    wid = sid * NC + cid
    pltpu.sync_copy(z_h, table.at[pl.ds(sid * RPT, RPT)])

    @pl.when(sid == 0)
    def _():
        pltpu.sync_copy(z_h.at[pl.ds(0, TAIL)], table.at[pl.ds(NS * RPT, TAIL)])

    plsc.subcore_barrier()

    def body(i, carry):
        ch = wid + i * NW

        @pl.when(ch < NCHUNK)
        def _():
            base = ch * C
            pltpu.sync_copy(b2revb_h.at[pl.ds(base, C)], ir)
            pltpu.async_copy(b2a_h.at[ir], idst, sem).wait()
            pltpu.sync_copy(idst, dest_h.at[pl.ds(base, C)])
            pltpu.sync_copy(h_h.at[pl.ds(base, C)], hbuf)
            pltpu.sync_copy(hbuf, table.at[idst], add=True)

        return carry

    lax.fori_loop(0, CPW, body, 0)
    plsc.subcore_barrier()
    rows = pl.ds(sid * RPT, RPT)
    pltpu.sync_copy(table.at[rows], spart_h.at[cid].at[rows])

    @pl.when(sid == 0)
    def _():
        tail = pl.ds(NS * RPT, TAIL)
        pltpu.sync_copy(table.at[tail], spart_h.at[cid].at[tail])


_sc_first = functools.partial(
    pl.kernel,
    out_type=(jax.ShapeDtypeStruct((NE,), jnp.int32),
              jax.ShapeDtypeStruct((NC, NA, H), jnp.float32)),
    mesh=_MESH,
    scratch_types=[
        pltpu.VMEM_SHARED((NA, H), jnp.float32),
        pltpu.VMEM((C, H), jnp.float32),
        pltpu.VMEM((C,), jnp.int32),
        pltpu.VMEM((C,), jnp.int32),
        pltpu.SemaphoreType.DMA,
    ],
)(_sc_first_body)


def _sc_step_body(h_h, sw_h, hw_h, b2a_h, b2revb_h, dest_h, z_h,
                  hnew_h, spart_h,
                  table,
                  hbuf0, hbuf1, obuf0, obuf1, swbuf0, swbuf1,
                  hwbuf0, hwbuf1, ia0, ia1, ir0, ir1, id0, id1,
                  si0, si1, sg0, sg1, so0, so1, sd0, sd1):
    cid = lax.axis_index("c")
    sid = lax.axis_index("s")
    wid = sid * NC + cid
    base0 = wid * CPWS * CS
    pltpu.sync_copy(z_h, table.at[pl.ds(sid * RPT, RPT)])

    @pl.when(sid == 0)
    def _():
        pltpu.sync_copy(z_h.at[pl.ds(0, TAIL)], table.at[pl.ds(NS * RPT, TAIL)])

    plsc.subcore_barrier()

    slot = [
        dict(h=hbuf0, o=obuf0, sw=swbuf0, hw=hwbuf0, ia=ia0, ir=ir0,
             idst=id0, si=si0, sg=sg0, so=so0, sd=sd0),
        dict(h=hbuf1, o=obuf1, sw=swbuf1, hw=hwbuf1, ia=ia1, ir=ir1,
             idst=id1, si=si1, sg=sg1, so=so1, sd=sd1),
    ]

    def prefetch(i, s):
        b = base0 + i * CS
        pltpu.async_copy(b2a_h.at[pl.ds(b, CS)], s["ia"], s["si"])
        pltpu.async_copy(b2revb_h.at[pl.ds(b, CS)], s["ir"], s["si"])
        pltpu.async_copy(h_h.at[pl.ds(b, CS)], s["h"], s["si"])

    def wait_in(i, s):
        b = base0 + i * CS
        pltpu.make_async_copy(b2a_h.at[pl.ds(b, CS)], s["ia"], s["si"]).wait()
        pltpu.make_async_copy(b2revb_h.at[pl.ds(b, CS)], s["ir"], s["si"]).wait()
        pltpu.make_async_copy(h_h.at[pl.ds(b, CS)], s["h"], s["si"]).wait()

    def idst_load(i, s):
        pltpu.async_copy(dest_h.at[pl.ds(base0 + i * CS, CS)], s["idst"], s["sd"])

    def wait_idst(i, s):
        pltpu.make_async_copy(dest_h.at[pl.ds(base0 + i * CS, CS)],
                              s["idst"], s["sd"]).wait()

    def gathers(s):
        pltpu.async_copy(sw_h.at[s["ia"]], s["sw"], s["sg"])
        pltpu.async_copy(hw_h.at[s["ir"]], s["hw"], s["sg"])

    def wait_g(s):
        pltpu.make_async_copy(sw_h.at[s["ia"]], s["sw"], s["sg"]).wait()
        pltpu.make_async_copy(hw_h.at[s["ir"]], s["hw"], s["sg"]).wait()

    def writeback(i, s):
        b = base0 + i * CS
        pltpu.async_copy(s["o"], hnew_h.at[pl.ds(b, CS)], s["so"])
        pltpu.sync_copy(s["o"], table.at[s["idst"]], add=True)

    def wait_out(i, s):
        b = base0 + i * CS
        pltpu.make_async_copy(s["o"], hnew_h.at[pl.ds(b, CS)], s["so"]).wait()

    def compute(s):
        hb, ob, sb, wb = s["h"], s["o"], s["sw"], s["hw"]

        def comb(r, cc):
            for k in range(H // 16):
                sl = pl.ds(k * 16, 16)
                ob[r, sl] = jnp.maximum(hb[r, sl] + sb[r, sl] - wb[r, sl], 0.0)
            return cc

        lax.fori_loop(0, CS, comb, 0)

    def emit(i, sl, last):
        if not last:
            other = slot[1] if sl is slot[0] else slot[0]
            wait_in(i + 1, other)
            gathers(other)
        wait_g(sl)

        def drain():
            wait_out(i - 2, sl)
            idst_load(i, sl)

        if isinstance(i, int):
            if i >= 2:
                drain()
        else:
            pl.when(i >= 2)(drain)
        compute(sl)
        wait_idst(i, sl)
        writeback(i, sl)
        if not last:
            def pre():
                prefetch(i + 2, sl)

            if isinstance(i, int):
                if i + 2 < CPWS:
                    pre()
            else:
                pl.when(i + 2 < CPWS)(pre)

    prefetch(0, slot[0])
    idst_load(0, slot[0])
    prefetch(1, slot[1])
    idst_load(1, slot[1])
    wait_in(0, slot[0])
    gathers(slot[0])

    def pair(p, carry):
        emit(2 * p, slot[0], last=False)
        emit(2 * p + 1, slot[1], last=False)
        return carry

    lax.fori_loop(0, CPWS // 2 - 1, pair, 0)
    emit(CPWS - 2, slot[0], last=False)
    emit(CPWS - 1, slot[1], last=True)

    wait_out(CPWS - 2, slot[(CPWS - 2) % 2])
    wait_out(CPWS - 1, slot[(CPWS - 1) % 2])

    plsc.subcore_barrier()
    rows = pl.ds(sid * RPT, RPT)
    pltpu.sync_copy(table.at[rows], spart_h.at[cid].at[rows])

    @pl.when(sid == 0)
    def _():
        tail = pl.ds(NS * RPT, TAIL)
        pltpu.sync_copy(table.at[tail], spart_h.at[cid].at[tail])


_sc_step = functools.partial(
    pl.kernel,
    out_type=(jax.ShapeDtypeStruct((NE, H), jnp.float32),
              jax.ShapeDtypeStruct((NC, NA, H), jnp.float32)),
    mesh=_MESH,
    scratch_types=(
        [pltpu.VMEM_SHARED((NA, H), jnp.float32)]
        + [pltpu.VMEM((CS, H), jnp.float32)] * 8
        + [pltpu.VMEM((CS,), jnp.int32)] * 6
        + [pltpu.SemaphoreType.DMA] * 8
    ),
)(_sc_step_body)


def _tc_in_body(fb, win, bin_, wmsg, h_o, hw_o):
    h = jnp.maximum(
        jnp.dot(fb[...], win[...], preferred_element_type=jnp.float32)
        + bin_[...], 0.0)
    h_o[...] = h
    hw_o[...] = jnp.dot(h, wmsg[...], preferred_element_type=jnp.float32)


def _tc_in(f_bonds, W_in_w, bin2, W_msg_w):
    R = 1280
    return pl.pallas_call(
        _tc_in_body,
        grid=(NE // R,),
        in_specs=[pl.BlockSpec((R, DIN), lambda i: (i, 0)),
                  pl.BlockSpec((DIN, H), lambda i: (0, 0)),
                  pl.BlockSpec((1, H), lambda i: (0, 0)),
                  pl.BlockSpec((H, H), lambda i: (0, 0))],
        out_specs=[pl.BlockSpec((R, H), lambda i: (i, 0)),
                   pl.BlockSpec((R, H), lambda i: (i, 0))],
        out_shape=[jax.ShapeDtypeStruct((NE, H), jnp.float32),
                   jax.ShapeDtypeStruct((NE, H), jnp.float32)],
    )(f_bonds, W_in_w, bin2, W_msg_w)


def _tc_hw_body(h, wmsg, hw_o):
    hw_o[...] = jnp.dot(h[...], wmsg[...], preferred_element_type=jnp.float32)


def _tc_hw(h, W_msg_w):
    R = 2000
    return pl.pallas_call(
        _tc_hw_body,
        grid=(NE // R,),
        in_specs=[pl.BlockSpec((R, H), lambda i: (i, 0)),
                  pl.BlockSpec((H, H), lambda i: (0, 0))],
        out_specs=pl.BlockSpec((R, H), lambda i: (i, 0)),
        out_shape=jax.ShapeDtypeStruct((NE, H), jnp.float32),
    )(h, W_msg_w)


def _tc_sw_body(s0, s1, wmsg, bmsg, sw_o):
    s = s0[...] + s1[...]
    sw_o[...] = (jnp.dot(s, wmsg[...], preferred_element_type=jnp.float32)
                 + bmsg[...])


def _tc_sw(s0, s1, W_msg_w, bmsg2):
    return pl.pallas_call(
        _tc_sw_body,
        out_shape=jax.ShapeDtypeStruct((NA, H), jnp.float32),
    )(s0, s1, W_msg_w, bmsg2)


def _tc_atom_body(s0, s1, fa, was, waf, bat, nw, nb, hatom_o, node_o):
    s = s0[...] + s1[...]
    h_atom = jnp.maximum(
        jnp.dot(s, was[...], preferred_element_type=jnp.float32)
        + jnp.dot(fa[...], waf[...], preferred_element_type=jnp.float32)
        + bat[...], 0.0)
    hatom_o[...] = h_atom
    node_o[...] = (jnp.dot(h_atom, nw[...], preferred_element_type=jnp.float32)
                   + nb[...])


def _tc_atom(s0, s1, f_atoms, wa_s, wa_f, bat2, node_w, nb2):
    return pl.pallas_call(
        _tc_atom_body,
        out_shape=[jax.ShapeDtypeStruct((NA, H), jnp.float32),
                   jax.ShapeDtypeStruct((NA, H), jnp.float32)],
    )(s0, s1, f_atoms, wa_s, wa_f, bat2, node_w, nb2)


def _tc_edge_body(h, ew, eb, out):
    out[...] = (jnp.dot(h[...], ew[...], preferred_element_type=jnp.float32)
                + eb[...])


def _tc_edge(h, edge_w, eb2):
    R = 2000
    return pl.pallas_call(
        _tc_edge_body,
        grid=(NE // R,),
        in_specs=[pl.BlockSpec((R, H), lambda i: (i, 0)),
                  pl.BlockSpec((H, DIN), lambda i: (0, 0)),
                  pl.BlockSpec((1, DIN), lambda i: (0, 0))],
        out_specs=pl.BlockSpec((R, DIN), lambda i: (i, 0)),
        out_shape=jax.ShapeDtypeStruct((NE, DIN), jnp.float32),
    )(h, edge_w, eb2)


def kernel(f_atoms, f_bonds, a2b, b2a, b2revb,
           W_in_w, W_in_b, W_msg_w, W_msg_b,
           W_atom_w, W_atom_b, node_w, node_b, edge_w, edge_b):
    del a2b
    b2a = b2a.astype(jnp.int32)
    b2revb = b2revb.astype(jnp.int32)
    bin2 = W_in_b.reshape(1, H)
    bmsg2 = W_msg_b.reshape(1, H)
    bat2 = W_atom_b.reshape(1, H)
    nb2 = node_b.reshape(1, H)
    eb2 = edge_b.reshape(1, DIN)
    wa_s = W_atom_w[0:H, :]
    wa_f = W_atom_w[H:2 * H, :]
    zrows = jnp.zeros((RPT, H), jnp.float32)

    h, hw = _tc_in(f_bonds, W_in_w, bin2, W_msg_w)
    dest, spart = _sc_first(b2a, b2revb, h, zrows)
    for step in range(3):
        sw = _tc_sw(spart[0], spart[1], W_msg_w, bmsg2)
        h, spart = _sc_step(h, sw, hw, b2a, b2revb, dest, zrows)
        if step < 2:
            hw = _tc_hw(h, W_msg_w)

    h_atom, node_pred = _tc_atom(spart[0], spart[1], f_atoms,
                                 wa_s, wa_f, bat2, node_w, nb2)
    edge_pred = _tc_edge(h, edge_w, eb2)
    return (node_pred, edge_pred, h_atom)


# speedup vs baseline: 1.9385x; 1.0309x over previous
import functools

import jax
import jax.numpy as jnp
from jax import lax
from jax.experimental import pallas as pl
from jax.experimental.pallas import tpu as pltpu
from jax.experimental.pallas import tpu_sc as plsc

NE = 320000
NA = 10000
H = 128
DIN = 144
NC, NS = 2, 16
NW = NC * NS
C = 128
NCHUNK = NE // C
CPW = (NCHUNK + NW - 1) // NW
CS = 40
NCHUNKS = NE // CS
CPWS = NCHUNKS // NW
RPT = 624
TAIL = NA - RPT * NS

_MESH = plsc.VectorSubcoreMesh(core_axis_name="c", subcore_axis_name="s")


def _sc_first_body(b2a_h, b2revb_h, h_h, z_h, dest_h, spart_h,
                   table, hbuf, ir, idst, sem, semh):
    cid = lax.axis_index("c")
    sid = lax.axis_index("s")
    wid = sid * NC + cid
    pltpu.sync_copy(z_h, table.at[pl.ds(sid * RPT, RPT)])

    @pl.when(sid == 0)
    def _():
        pltpu.sync_copy(z_h.at[pl.ds(0, TAIL)], table.at[pl.ds(NS * RPT, TAIL)])

    plsc.subcore_barrier()

    def body(i, carry):
        ch = wid + i * NW

        @pl.when(ch < NCHUNK)
        def _():
            base = ch * C
            cph = pltpu.async_copy(h_h.at[pl.ds(base, C)], hbuf, semh)
            pltpu.sync_copy(b2revb_h.at[pl.ds(base, C)], ir)
            pltpu.async_copy(b2a_h.at[ir], idst, sem).wait()
            pltpu.sync_copy(idst, dest_h.at[pl.ds(base, C)])
            cph.wait()
            pltpu.sync_copy(hbuf, table.at[idst], add=True)

        return carry

    lax.fori_loop(0, CPW, body, 0)
    plsc.subcore_barrier()
    rows = pl.ds(sid * RPT, RPT)
    pltpu.sync_copy(table.at[rows], spart_h.at[cid].at[rows])

    @pl.when(sid == 0)
    def _():
        tail = pl.ds(NS * RPT, TAIL)
        pltpu.sync_copy(table.at[tail], spart_h.at[cid].at[tail])


_sc_first = functools.partial(
    pl.kernel,
    out_type=(jax.ShapeDtypeStruct((NE,), jnp.int32),
              jax.ShapeDtypeStruct((NC, NA, H), jnp.float32)),
    mesh=_MESH,
    scratch_types=[
        pltpu.VMEM_SHARED((NA, H), jnp.float32),
        pltpu.VMEM((C, H), jnp.float32),
        pltpu.VMEM((C,), jnp.int32),
        pltpu.VMEM((C,), jnp.int32),
        pltpu.SemaphoreType.DMA,
        pltpu.SemaphoreType.DMA,
    ],
)(_sc_first_body)


def _sc_step_body(h_h, sw_h, hw_h, b2a_h, b2revb_h, dest_h, z_h,
                  hnew_h, spart_h,
                  table,
                  hbuf0, hbuf1, obuf0, obuf1, swbuf0, swbuf1,
                  hwbuf0, hwbuf1, ia0, ia1, ir0, ir1, id0, id1,
                  si0, si1, sg0, sg1, so0, so1, sd0, sd1):
    cid = lax.axis_index("c")
    sid = lax.axis_index("s")
    wid = sid * NC + cid
    base0 = wid * CPWS * CS
    pltpu.sync_copy(z_h, table.at[pl.ds(sid * RPT, RPT)])

    @pl.when(sid == 0)
    def _():
        pltpu.sync_copy(z_h.at[pl.ds(0, TAIL)], table.at[pl.ds(NS * RPT, TAIL)])

    plsc.subcore_barrier()

    slot = [
        dict(h=hbuf0, o=obuf0, sw=swbuf0, hw=hwbuf0, ia=ia0, ir=ir0,
             idst=id0, si=si0, sg=sg0, so=so0, sd=sd0),
        dict(h=hbuf1, o=obuf1, sw=swbuf1, hw=hwbuf1, ia=ia1, ir=ir1,
             idst=id1, si=si1, sg=sg1, so=so1, sd=sd1),
    ]

    def prefetch(i, s):
        b = base0 + i * CS
        pltpu.async_copy(b2a_h.at[pl.ds(b, CS)], s["ia"], s["si"])
        pltpu.async_copy(b2revb_h.at[pl.ds(b, CS)], s["ir"], s["si"])
        pltpu.async_copy(h_h.at[pl.ds(b, CS)], s["h"], s["si"])

    def wait_in(i, s):
        b = base0 + i * CS
        pltpu.make_async_copy(b2a_h.at[pl.ds(b, CS)], s["ia"], s["si"]).wait()
        pltpu.make_async_copy(b2revb_h.at[pl.ds(b, CS)], s["ir"], s["si"]).wait()
        pltpu.make_async_copy(h_h.at[pl.ds(b, CS)], s["h"], s["si"]).wait()

    def idst_load(i, s):
        pltpu.async_copy(dest_h.at[pl.ds(base0 + i * CS, CS)], s["idst"], s["sd"])

    def wait_idst(i, s):
        pltpu.make_async_copy(dest_h.at[pl.ds(base0 + i * CS, CS)],
                              s["idst"], s["sd"]).wait()

    def gathers(s):
        pltpu.async_copy(sw_h.at[s["ia"]], s["sw"], s["sg"])
        pltpu.async_copy(hw_h.at[s["ir"]], s["hw"], s["sg"])

    def wait_g(s):
        pltpu.make_async_copy(sw_h.at[s["ia"]], s["sw"], s["sg"]).wait()
        pltpu.make_async_copy(hw_h.at[s["ir"]], s["hw"], s["sg"]).wait()

    def writeback(i, s):
        b = base0 + i * CS
        pltpu.async_copy(s["o"], hnew_h.at[pl.ds(b, CS)], s["so"])
        pltpu.sync_copy(s["o"], table.at[s["idst"]], add=True)

    def wait_out(i, s):
        b = base0 + i * CS
        pltpu.make_async_copy(s["o"], hnew_h.at[pl.ds(b, CS)], s["so"]).wait()

    def compute(s):
        hb, ob, sb, wb = s["h"], s["o"], s["sw"], s["hw"]

        def comb(r, cc):
            for k in range(H // 16):
                sl = pl.ds(k * 16, 16)
                ob[r, sl] = jnp.maximum(hb[r, sl] + sb[r, sl] - wb[r, sl], 0.0)
            return cc

        lax.fori_loop(0, CS, comb, 0)

    def emit(i, sl, last):
        if not last:
            other = slot[1] if sl is slot[0] else slot[0]
            wait_in(i + 1, other)
            gathers(other)
        wait_g(sl)

        def drain():
            wait_out(i - 2, sl)
            idst_load(i, sl)

        if isinstance(i, int):
            if i >= 2:
                drain()
        else:
            pl.when(i >= 2)(drain)
        compute(sl)
        wait_idst(i, sl)
        writeback(i, sl)
        if not last:
            def pre():
                prefetch(i + 2, sl)

            if isinstance(i, int):
                if i + 2 < CPWS:
                    pre()
            else:
                pl.when(i + 2 < CPWS)(pre)

    prefetch(0, slot[0])
    idst_load(0, slot[0])
    prefetch(1, slot[1])
    idst_load(1, slot[1])
    wait_in(0, slot[0])
    gathers(slot[0])

    def pair(p, carry):
        emit(2 * p, slot[0], last=False)
        emit(2 * p + 1, slot[1], last=False)
        return carry

    lax.fori_loop(0, CPWS // 2 - 1, pair, 0)
    emit(CPWS - 2, slot[0], last=False)
    emit(CPWS - 1, slot[1], last=True)

    wait_out(CPWS - 2, slot[(CPWS - 2) % 2])
    wait_out(CPWS - 1, slot[(CPWS - 1) % 2])

    plsc.subcore_barrier()
    rows = pl.ds(sid * RPT, RPT)
    pltpu.sync_copy(table.at[rows], spart_h.at[cid].at[rows])

    @pl.when(sid == 0)
    def _():
        tail = pl.ds(NS * RPT, TAIL)
        pltpu.sync_copy(table.at[tail], spart_h.at[cid].at[tail])


_sc_step = functools.partial(
    pl.kernel,
    out_type=(jax.ShapeDtypeStruct((NE, H), jnp.float32),
              jax.ShapeDtypeStruct((NC, NA, H), jnp.float32)),
    mesh=_MESH,
    scratch_types=(
        [pltpu.VMEM_SHARED((NA, H), jnp.float32)]
        + [pltpu.VMEM((CS, H), jnp.float32)] * 8
        + [pltpu.VMEM((CS,), jnp.int32)] * 6
        + [pltpu.SemaphoreType.DMA] * 8
    ),
)(_sc_step_body)


def _tc_in_body(fb, win, bin_, wmsg, h_o, hw_o):
    h = jnp.maximum(
        jnp.dot(fb[...], win[...], preferred_element_type=jnp.float32)
        + bin_[...], 0.0)
    h_o[...] = h
    hw_o[...] = jnp.dot(h, wmsg[...], preferred_element_type=jnp.float32)


def _tc_in(f_bonds, W_in_w, bin2, W_msg_w):
    R = 1280
    return pl.pallas_call(
        _tc_in_body,
        grid=(NE // R,),
        in_specs=[pl.BlockSpec((R, DIN), lambda i: (i, 0)),
                  pl.BlockSpec((DIN, H), lambda i: (0, 0)),
                  pl.BlockSpec((1, H), lambda i: (0, 0)),
                  pl.BlockSpec((H, H), lambda i: (0, 0))],
        out_specs=[pl.BlockSpec((R, H), lambda i: (i, 0)),
                   pl.BlockSpec((R, H), lambda i: (i, 0))],
        out_shape=[jax.ShapeDtypeStruct((NE, H), jnp.float32),
                   jax.ShapeDtypeStruct((NE, H), jnp.float32)],
    )(f_bonds, W_in_w, bin2, W_msg_w)


def _tc_hw_body(h, wmsg, hw_o):
    hw_o[...] = jnp.dot(h[...], wmsg[...], preferred_element_type=jnp.float32)


def _tc_hw(h, W_msg_w):
    R = 2000
    return pl.pallas_call(
        _tc_hw_body,
        grid=(NE // R,),
        in_specs=[pl.BlockSpec((R, H), lambda i: (i, 0)),
                  pl.BlockSpec((H, H), lambda i: (0, 0))],
        out_specs=pl.BlockSpec((R, H), lambda i: (i, 0)),
        out_shape=jax.ShapeDtypeStruct((NE, H), jnp.float32),
    )(h, W_msg_w)


def _tc_sw_body(s0, s1, wmsg, bmsg, sw_o):
    s = s0[...] + s1[...]
    sw_o[...] = (jnp.dot(s, wmsg[...], preferred_element_type=jnp.float32)
                 + bmsg[...])


def _tc_sw(s0, s1, W_msg_w, bmsg2):
    return pl.pallas_call(
        _tc_sw_body,
        out_shape=jax.ShapeDtypeStruct((NA, H), jnp.float32),
    )(s0, s1, W_msg_w, bmsg2)


def _tc_atom_body(s0, s1, fa, was, waf, bat, nw, nb, hatom_o, node_o):
    s = s0[...] + s1[...]
    h_atom = jnp.maximum(
        jnp.dot(s, was[...], preferred_element_type=jnp.float32)
        + jnp.dot(fa[...], waf[...], preferred_element_type=jnp.float32)
        + bat[...], 0.0)
    hatom_o[...] = h_atom
    node_o[...] = (jnp.dot(h_atom, nw[...], preferred_element_type=jnp.float32)
                   + nb[...])


def _tc_atom(s0, s1, f_atoms, wa_s, wa_f, bat2, node_w, nb2):
    return pl.pallas_call(
        _tc_atom_body,
        out_shape=[jax.ShapeDtypeStruct((NA, H), jnp.float32),
                   jax.ShapeDtypeStruct((NA, H), jnp.float32)],
    )(s0, s1, f_atoms, wa_s, wa_f, bat2, node_w, nb2)


def _tc_edge_body(h, ew, eb, out):
    out[...] = (jnp.dot(h[...], ew[...], preferred_element_type=jnp.float32)
                + eb[...])


def _tc_edge(h, edge_w, eb2):
    R = 2000
    return pl.pallas_call(
        _tc_edge_body,
        grid=(NE // R,),
        in_specs=[pl.BlockSpec((R, H), lambda i: (i, 0)),
                  pl.BlockSpec((H, DIN), lambda i: (0, 0)),
                  pl.BlockSpec((1, DIN), lambda i: (0, 0))],
        out_specs=pl.BlockSpec((R, DIN), lambda i: (i, 0)),
        out_shape=jax.ShapeDtypeStruct((NE, DIN), jnp.float32),
    )(h, edge_w, eb2)


def kernel(f_atoms, f_bonds, a2b, b2a, b2revb,
           W_in_w, W_in_b, W_msg_w, W_msg_b,
           W_atom_w, W_atom_b, node_w, node_b, edge_w, edge_b):
    del a2b
    b2a = b2a.astype(jnp.int32)
    b2revb = b2revb.astype(jnp.int32)
    bin2 = W_in_b.reshape(1, H)
    bmsg2 = W_msg_b.reshape(1, H)
    bat2 = W_atom_b.reshape(1, H)
    nb2 = node_b.reshape(1, H)
    eb2 = edge_b.reshape(1, DIN)
    wa_s = W_atom_w[0:H, :]
    wa_f = W_atom_w[H:2 * H, :]
    zrows = jnp.zeros((RPT, H), jnp.float32)

    h, hw = _tc_in(f_bonds, W_in_w, bin2, W_msg_w)
    dest, spart = _sc_first(b2a, b2revb, h, zrows)
    for step in range(3):
        sw = _tc_sw(spart[0], spart[1], W_msg_w, bmsg2)
        h, spart = _sc_step(h, sw, hw, b2a, b2revb, dest, zrows)
        if step < 2:
            hw = _tc_hw(h, W_msg_w)

    h_atom, node_pred = _tc_atom(spart[0], spart[1], f_atoms,
                                 wa_s, wa_f, bat2, node_w, nb2)
    edge_pred = _tc_edge(h, edge_w, eb2)
    return (node_pred, edge_pred, h_atom)
